# baseline (speedup 1.0000x reference)
import functools

import jax
import jax.numpy as jnp
import numpy as np
from jax.experimental import pallas as pl
from jax.experimental.pallas import tpu as pltpu

_PERCENT = 0.1
_MAX_FINITE_HI = 0x7F7F
_CT = 16


def _mask_body(w_ref, mask_ref, sal_ref, *, K, D):
    w = w_ref[...]
    rw = jnp.abs(w.astype(jnp.bfloat16).astype(jnp.float32))
    sal_ref[...] = jax.lax.shift_right_logical(
        jax.lax.bitcast_convert_type(rw, jnp.int32), 16
    ).astype(jnp.int16)

    def bisect(_, lohi):
        lo, hi = lohi
        mid = lo + (hi - lo + 1) // 2
        cnt = jnp.sum((sal_ref[...] >= mid.astype(jnp.int16))
                      .astype(jnp.int32), axis=1, keepdims=True)
        take = cnt >= K
        return (jnp.where(take, mid, lo), jnp.where(take, hi, mid - 1))

    thresh, _ = jax.lax.fori_loop(
        0, 15, bisect,
        (jnp.zeros((_CT, 1), jnp.int32),
         jnp.full((_CT, 1), _MAX_FINITE_HI, jnp.int32)),
    )

    thresh16 = thresh.astype(jnp.int16)
    n_greater = jnp.sum((sal_ref[...] > thresh16).astype(jnp.int32), axis=1,
                        keepdims=True)
    need = K - n_greater

    def bisect_idx(_, lohi):
        lo2, hi2 = lohi
        mid = (lo2 + hi2) // 2
        bits = sal_ref[...]
        idx = jax.lax.broadcasted_iota(jnp.int32, bits.shape, 1)
        cm = jnp.sum(((bits == thresh16) & (idx < mid)).astype(jnp.int32),
                     axis=1, keepdims=True)
        take = cm >= need
        return (jnp.where(take, lo2, mid), jnp.where(take, mid, hi2))

    _, mcut = jax.lax.fori_loop(
        0, 18, bisect_idx,
        (jnp.zeros((_CT, 1), jnp.int32), jnp.full((_CT, 1), D, jnp.int32)),
    )

    bits = sal_ref[...]
    idx = jax.lax.broadcasted_iota(jnp.int32, bits.shape, 1)
    sel = (bits > thresh16) | ((bits == thresh16) & (idx < mcut))
    mask_ref[...] = sel.astype(jnp.bfloat16)


def _select_body(x_ref, m_ref, p_ref, oh_ref, out_ref):
    x = x_ref[...]
    pb = p_ref[...]
    dn = (((1,), (0,)), ((), ()))
    f32 = jnp.float32
    xh = x.astype(jnp.bfloat16)
    xl = (x - xh.astype(f32)).astype(jnp.bfloat16)
    xs = (jax.lax.dot_general(xh, pb, dn, preferred_element_type=f32)
          + jax.lax.dot_general(xl, pb, dn, preferred_element_type=f32))
    mb = jax.lax.dot_general(m_ref[...], oh_ref[...], (((0,), (0,)), ((), ())),
                             preferred_element_type=f32)
    out_ref[...] = jnp.where(mb != 0.0, xs, x)


def kernel(Data, Labels, W, perm):
    B = Data.shape[0]
    H, Wd, Ch = Data.shape[1], Data.shape[2], Data.shape[3]
    D = H * Wd * Ch
    C = W.shape[1]
    K = int(np.floor(_PERCENT * D))
    CH = 3072 if D % 3072 == 0 else D
    NC = D // CH

    X = Data.transpose(1, 3, 2, 0).reshape(D, B)
    Wt = W.T

    mask = pl.pallas_call(
        functools.partial(_mask_body, K=K, D=D),
        grid=((C + _CT - 1) // _CT,),
        in_specs=[pl.BlockSpec((_CT, D), lambda c: (c, 0))],
        out_specs=pl.BlockSpec((_CT, D), lambda c: (c, 0)),
        out_shape=jax.ShapeDtypeStruct((C, D), jnp.bfloat16),
        scratch_shapes=[pltpu.VMEM((_CT, D), jnp.int16)],
    )(Wt)

    maskP = (mask.reshape(C, H, Wd, Ch).transpose(0, 1, 3, 2)
             .reshape(C, D))

    ar = jnp.arange(B, dtype=jnp.int32)
    P = (ar[:, None] == perm[None, :]).astype(jnp.bfloat16)
    oh = (jnp.arange(C, dtype=jnp.int32)[:, None] == Labels[None, :]
          ).astype(jnp.bfloat16)

    outX = pl.pallas_call(
        _select_body,
        grid=(NC,),
        in_specs=[
            pl.BlockSpec((CH, B), lambda t: (t, 0)),
            pl.BlockSpec((C, CH), lambda t: (0, t)),
            pl.BlockSpec((B, B), lambda t: (0, 0)),
            pl.BlockSpec((C, B), lambda t: (0, 0)),
        ],
        out_specs=pl.BlockSpec((CH, B), lambda t: (t, 0)),
        out_shape=jax.ShapeDtypeStruct((D, B), jnp.float32),
    )(X, maskP, P, oh)

    return outX.reshape(H, Ch, Wd, B).transpose(3, 0, 2, 1)


# speedup vs baseline: 1.1125x; 1.1125x over previous
import functools

import jax
import jax.numpy as jnp
import numpy as np
from jax.experimental import pallas as pl
from jax.experimental.pallas import tpu as pltpu

_PERCENT = 0.1
_MAX_FINITE_HI = 0x7F7F
_CT = 16


def _mask_body(w_ref, mask_ref, sal_ref, *, K, D):
    w = w_ref[...]
    rw = jnp.abs(w.astype(jnp.bfloat16).astype(jnp.float32))
    sal_ref[...] = jax.lax.shift_right_logical(
        jax.lax.bitcast_convert_type(rw, jnp.int32), 16
    )

    def bisect(_, lohi):
        lo, hi = lohi
        mid = lo + (hi - lo + 1) // 2
        cnt = jnp.sum((sal_ref[...] >= mid).astype(jnp.int32), axis=1,
                      keepdims=True)
        take = cnt >= K
        return (jnp.where(take, mid, lo), jnp.where(take, hi, mid - 1))

    thresh, _ = jax.lax.fori_loop(
        0, 15, bisect,
        (jnp.zeros((_CT, 1), jnp.int32),
         jnp.full((_CT, 1), _MAX_FINITE_HI, jnp.int32)),
    )

    n_greater = jnp.sum((sal_ref[...] > thresh).astype(jnp.int32), axis=1,
                        keepdims=True)
    need = K - n_greater

    def bisect_idx(_, lohi):
        lo2, hi2 = lohi
        mid = (lo2 + hi2) // 2
        bits = sal_ref[...]
        idx = jax.lax.broadcasted_iota(jnp.int32, bits.shape, 1)
        cm = jnp.sum(((bits == thresh) & (idx < mid)).astype(jnp.int32),
                     axis=1, keepdims=True)
        take = cm >= need
        return (jnp.where(take, lo2, mid), jnp.where(take, mid, hi2))

    _, mcut = jax.lax.fori_loop(
        0, 18, bisect_idx,
        (jnp.zeros((_CT, 1), jnp.int32), jnp.full((_CT, 1), D, jnp.int32)),
    )

    bits = sal_ref[...]
    idx = jax.lax.broadcasted_iota(jnp.int32, bits.shape, 1)
    sel = (bits > thresh) | ((bits == thresh) & (idx < mcut))
    mask_ref[...] = sel.astype(jnp.bfloat16)


def _select_body(x_ref, m_ref, p_ref, oh_ref, out_ref):
    x = x_ref[...]
    pb = p_ref[...]
    dn = (((1,), (0,)), ((), ()))
    f32 = jnp.float32
    xh = x.astype(jnp.bfloat16)
    xl = (x - xh.astype(f32)).astype(jnp.bfloat16)
    xs = (jax.lax.dot_general(xh, pb, dn, preferred_element_type=f32)
          + jax.lax.dot_general(xl, pb, dn, preferred_element_type=f32))
    mb = jax.lax.dot_general(m_ref[...], oh_ref[...], (((0,), (0,)), ((), ())),
                             preferred_element_type=f32)
    out_ref[...] = jnp.where(mb != 0.0, xs, x)


def kernel(Data, Labels, W, perm):
    B = Data.shape[0]
    H, Wd, Ch = Data.shape[1], Data.shape[2], Data.shape[3]
    D = H * Wd * Ch
    C = W.shape[1]
    K = int(np.floor(_PERCENT * D))
    CH = 3072 if D % 3072 == 0 else D
    NC = D // CH

    X = Data.transpose(1, 3, 2, 0).reshape(D, B)
    Wt = W.T

    mask = pl.pallas_call(
        functools.partial(_mask_body, K=K, D=D),
        grid=((C + _CT - 1) // _CT,),
        in_specs=[pl.BlockSpec((_CT, D), lambda c: (c, 0))],
        out_specs=pl.BlockSpec((_CT, D), lambda c: (c, 0)),
        out_shape=jax.ShapeDtypeStruct((C, D), jnp.bfloat16),
        scratch_shapes=[pltpu.VMEM((_CT, D), jnp.int32)],
    )(Wt)

    maskP = (mask.reshape(C, H, Wd, Ch).transpose(0, 1, 3, 2)
             .reshape(C, D))

    ar = jnp.arange(B, dtype=jnp.int32)
    P = (ar[:, None] == perm[None, :]).astype(jnp.bfloat16)
    oh = (jnp.arange(C, dtype=jnp.int32)[:, None] == Labels[None, :]
          ).astype(jnp.bfloat16)

    outX = pl.pallas_call(
        _select_body,
        grid=(NC,),
        in_specs=[
            pl.BlockSpec((CH, B), lambda t: (t, 0)),
            pl.BlockSpec((C, CH), lambda t: (0, t)),
            pl.BlockSpec((B, B), lambda t: (0, 0)),
            pl.BlockSpec((C, B), lambda t: (0, 0)),
        ],
        out_specs=pl.BlockSpec((CH, B), lambda t: (t, 0)),
        out_shape=jax.ShapeDtypeStruct((D, B), jnp.float32),
    )(X, maskP, P, oh)

    return outX.reshape(H, Ch, Wd, B).transpose(3, 0, 2, 1)
